# Initial kernel scaffold; baseline (speedup 1.0000x reference)
import jax
import jax.numpy as jnp
from jax.experimental import pallas as pl


def kernel(x, edge_index, W1, b1, gamma, beta, W2, b2):
    raise NotImplementedError("write your pallas kernel here")


# speedup vs baseline: 32.5551x; 32.5551x over previous
import functools

import jax
import jax.numpy as jnp
from jax import lax
from jax.experimental import pallas as pl
from jax.experimental.pallas import tpu as pltpu
from jax.experimental.pallas import tpu_sc as plsc

N = 10000
E = 320000
D = 128
H = 16
C = 2

NW = 32
CH = 128
CPW = 80
EP = NW * CPW * CH
PAD = EP - E
ROWS2D = EP // CH
K_JUNK = 16
NP = N + K_JUNK

BLK = 2000
GRID = N // BLK

def _sc_deg_body(dst_hbm, zf_hbm, out_hbm, dst_v, ones_v, stage, acc):
    c = lax.axis_index("c")
    s = lax.axis_index("s")
    w = s * 2 + c

    @pl.when(s < 10)
    def _():
        pltpu.sync_copy(zf_hbm, stage)
        pltpu.sync_copy(stage, acc.at[pl.ds(s * 1000, 1000)])

    @pl.when(s == 10)
    def _():
        pltpu.sync_copy(zf_hbm, stage)
        pltpu.sync_copy(stage.at[pl.ds(0, K_JUNK)], acc.at[pl.ds(N, K_JUNK)])

    pltpu.sync_copy(dst_hbm.at[pl.ds(w * CPW, CPW)], dst_v)
    for i in range(CH // 16):
        ones_v[pl.ds(i * 16, 16)] = jnp.ones((16,), jnp.float32)
    plsc.subcore_barrier()

    def body(j, carry):
        pltpu.sync_copy(ones_v, acc.at[dst_v.at[j]], add=True)
        return carry

    lax.fori_loop(0, CPW, body, 0)
    plsc.subcore_barrier()

    @pl.when(s < 10)
    def _():
        pltpu.sync_copy(acc.at[pl.ds(s * 1000, 1000)], stage)
        pltpu.sync_copy(stage, out_hbm.at[c].at[s])


def _sc_agg_body(src_hbm, dst_hbm, g_hbm, z_hbm, out_hbm, src_v, dst_v, rows_v, stage, acc, sem):
    c = lax.axis_index("c")
    s = lax.axis_index("s")
    w = s * 2 + c

    @pl.when(s < 10)
    def _():
        pltpu.sync_copy(z_hbm, stage)
        pltpu.sync_copy(stage, acc.at[pl.ds(s * 1000, 1000)])

    @pl.when(s == 10)
    def _():
        pltpu.sync_copy(z_hbm, stage)
        pltpu.sync_copy(stage.at[pl.ds(0, K_JUNK)], acc.at[pl.ds(N, K_JUNK)])

    pltpu.sync_copy(src_hbm.at[pl.ds(w * CPW, CPW)], src_v)
    pltpu.sync_copy(dst_hbm.at[pl.ds(w * CPW, CPW)], dst_v)
    plsc.subcore_barrier()

    def body(j, carry):
        pltpu.async_copy(g_hbm.at[src_v.at[j]], rows_v, sem).wait()
        pltpu.sync_copy(rows_v, acc.at[dst_v.at[j]], add=True)
        return carry

    lax.fori_loop(0, CPW, body, 0)
    plsc.subcore_barrier()

    @pl.when(s < 10)
    def _():
        pltpu.sync_copy(acc.at[pl.ds(s * 1000, 1000)], stage)
        pltpu.sync_copy(stage, out_hbm.at[c].at[pl.ds(s * 1000, 1000)])


@functools.cache
def _sc_kernels():
    mesh = plsc.VectorSubcoreMesh(core_axis_name="c", subcore_axis_name="s",
                                  num_cores=2, num_subcores=16)
    params = pltpu.CompilerParams(use_tc_tiling_on_sc=False)
    sc_deg = pl.kernel(
        _sc_deg_body,
        compiler_params=params,
        out_type=jax.ShapeDtypeStruct((2, 10, 1000), jnp.float32),
        mesh=mesh,
        scratch_types=[
            pltpu.VMEM((CPW, CH), jnp.int32),
            pltpu.VMEM((CH,), jnp.float32),
            pltpu.VMEM((1000,), jnp.float32),
            pltpu.VMEM_SHARED((NP,), jnp.float32),
        ],
    )
    sc_agg = pl.kernel(
        _sc_agg_body,
        compiler_params=params,
        out_type=jax.ShapeDtypeStruct((2, N, H), jnp.float32),
        mesh=mesh,
        scratch_types=[
            pltpu.VMEM((CPW, CH), jnp.int32),
            pltpu.VMEM((CPW, CH), jnp.int32),
            pltpu.VMEM((CH, H), jnp.float32),
            pltpu.VMEM((1000, H), jnp.float32),
            pltpu.VMEM_SHARED((NP, H), jnp.float32),
            pltpu.SemaphoreType.DMA,
        ],
    )
    return sc_deg, sc_agg


def _tc1_body(x_ref, w1_ref, degt_ref, h1_ref, dinv_ref, g1_ref):
    h1 = jnp.dot(x_ref[...], w1_ref[...], preferred_element_type=jnp.float32)
    deg = degt_ref[:, 0:1] + degt_ref[:, 1:2] + 1.0
    dinv = lax.rsqrt(deg)
    h1_ref[...] = h1
    dinv_ref[...] = dinv
    g1_ref[...] = dinv * h1


def _tc2_body(aggA_ref, aggB_ref, h1_ref, dinv_ref, b1_ref, out1_ref, st_ref):
    i = pl.program_id(0)
    dv = dinv_ref[...]
    o = dv * (aggA_ref[...] + aggB_ref[...]) + dv * dv * h1_ref[...] + b1_ref[...]
    out1_ref[...] = o
    ps = jnp.sum(o, axis=0, keepdims=True)
    pq = jnp.sum(o * o, axis=0, keepdims=True)
    pblk = jnp.concatenate([ps, pq], axis=0)

    @pl.when(i == 0)
    def _():
        st_ref[...] = pblk

    @pl.when(i > 0)
    def _():
        st_ref[...] = st_ref[...] + pblk


def _tc3_body(out1_ref, st_ref, gamma_ref, beta_ref, w2_ref, dinv_ref,
              h2_ref, g2_ref):
    st = st_ref[...]
    mean = st[0:1, :] * (1.0 / N)
    var = st[1:2, :] * (1.0 / N) - mean * mean
    z = (out1_ref[...] - mean) * lax.rsqrt(var + 1e-5) * gamma_ref[...] + beta_ref[...]
    z = jnp.maximum(z, 0.0)
    h2 = jnp.dot(z, w2_ref[...], preferred_element_type=jnp.float32)
    h2_ref[...] = h2
    g2_ref[...] = dinv_ref[...] * h2


def _tc4_body(aggA_ref, aggB_ref, h2_ref, dinv_ref, b2_ref, out_ref):
    dv = dinv_ref[...]
    o = dv * (aggA_ref[...] + aggB_ref[...]) + dv * dv * h2_ref[...] + b2_ref[...]
    o0 = o[:, 0:1]
    o1 = o[:, 1:2]
    m = jnp.maximum(o0, o1)
    lse = m + jnp.log(jnp.exp(o0 - m) + jnp.exp(o1 - m))
    out_ref[...] = jnp.concatenate([o0, o1], axis=1) - lse


def _row_spec(width):
    return pl.BlockSpec((BLK, width), lambda i: (i, 0))


def _full_spec(shape):
    return pl.BlockSpec(shape, lambda i: (0,) * len(shape))


def kernel(x, edge_index, W1, b1, gamma, beta, W2, b2):
    src = edge_index[0].astype(jnp.int32)
    dst = edge_index[1].astype(jnp.int32)
    ar = jnp.arange(PAD, dtype=jnp.int32)
    src_p = jnp.concatenate([src, (ar * 131) % N])
    dst_p = jnp.concatenate([dst, N + (ar % K_JUNK)])
    src2d = src_p.reshape(ROWS2D, CH)
    dst2d = dst_p.reshape(ROWS2D, CH)

    zf = jnp.zeros((1000,), jnp.float32)
    z16 = jnp.zeros((1000, H), jnp.float32)

    sc_deg, sc_agg = _sc_kernels()

    degp = sc_deg(dst2d, zf)
    degt = degp.reshape(2, N).T

    h1, dinv, g1 = pl.pallas_call(
        _tc1_body,
        grid=(GRID,),
        in_specs=[_row_spec(D), _full_spec((D, H)), _row_spec(2)],
        out_specs=[_row_spec(H), _row_spec(1), _row_spec(H)],
        out_shape=[
            jax.ShapeDtypeStruct((N, H), jnp.float32),
            jax.ShapeDtypeStruct((N, 1), jnp.float32),
            jax.ShapeDtypeStruct((N, H), jnp.float32),
        ],
    )(x, W1, degt)

    agg1 = sc_agg(src2d, dst2d, g1, z16)

    out1, st = pl.pallas_call(
        _tc2_body,
        grid=(GRID,),
        in_specs=[_row_spec(H), _row_spec(H), _row_spec(H), _row_spec(1),
                  _full_spec((1, H))],
        out_specs=[_row_spec(H), _full_spec((2, H))],
        out_shape=[
            jax.ShapeDtypeStruct((N, H), jnp.float32),
            jax.ShapeDtypeStruct((2, H), jnp.float32),
        ],
    )(agg1[0], agg1[1], h1, dinv, b1.reshape(1, H))

    W2p = jnp.pad(W2, ((0, 0), (0, H - C)))
    h2, g2 = pl.pallas_call(
        _tc3_body,
        grid=(GRID,),
        in_specs=[_row_spec(H), _full_spec((2, H)), _full_spec((1, H)),
                  _full_spec((1, H)), _full_spec((H, H)), _row_spec(1)],
        out_specs=[_row_spec(H), _row_spec(H)],
        out_shape=[
            jax.ShapeDtypeStruct((N, H), jnp.float32),
            jax.ShapeDtypeStruct((N, H), jnp.float32),
        ],
    )(out1, st, gamma.reshape(1, H), beta.reshape(1, H), W2p, dinv)

    agg2 = sc_agg(src2d, dst2d, g2, z16)

    b2p = jnp.pad(b2, (0, H - C)).reshape(1, H)
    out = pl.pallas_call(
        _tc4_body,
        grid=(GRID,),
        in_specs=[_row_spec(H), _row_spec(H), _row_spec(H), _row_spec(1),
                  _full_spec((1, H))],
        out_specs=_row_spec(C),
        out_shape=jax.ShapeDtypeStruct((N, C), jnp.float32),
    )(agg2[0], agg2[1], h2, dinv, b2p)
    return out


# speedup vs baseline: 45.0669x; 1.3843x over previous
import functools

import jax
import jax.numpy as jnp
from jax import lax
from jax.experimental import pallas as pl
from jax.experimental.pallas import tpu as pltpu
from jax.experimental.pallas import tpu_sc as plsc

N = 10000
E = 320000
D = 128
H = 16
C = 2

NW = 32
CH = 128
CPW = 80
EP = NW * CPW * CH
PAD = EP - E
ROWS2D = EP // CH
K_JUNK = 16
NP = N + K_JUNK

BLK = 2000
GRID = N // BLK

def _sc_deg_body(dst_hbm, zf_hbm, out_hbm, dst_v, ones_v, stage, acc):
    c = lax.axis_index("c")
    s = lax.axis_index("s")
    w = s * 2 + c

    @pl.when(s < 10)
    def _():
        pltpu.sync_copy(zf_hbm, stage)
        pltpu.sync_copy(stage, acc.at[pl.ds(s * 1000, 1000)])

    @pl.when(s == 10)
    def _():
        pltpu.sync_copy(zf_hbm, stage)
        pltpu.sync_copy(stage.at[pl.ds(0, K_JUNK)], acc.at[pl.ds(N, K_JUNK)])

    pltpu.sync_copy(dst_hbm.at[pl.ds(w * CPW, CPW)], dst_v)
    for i in range(CH // 16):
        ones_v[pl.ds(i * 16, 16)] = jnp.ones((16,), jnp.float32)
    plsc.subcore_barrier()

    def body(j, carry):
        pltpu.sync_copy(ones_v, acc.at[dst_v.at[j]], add=True)
        return carry

    lax.fori_loop(0, CPW, body, 0)
    plsc.subcore_barrier()

    @pl.when(s < 10)
    def _():
        pltpu.sync_copy(acc.at[pl.ds(s * 1000, 1000)], stage)
        pltpu.sync_copy(stage, out_hbm.at[c].at[s])


def _sc_agg_body(src_hbm, dst_hbm, g_hbm, z_hbm, out_hbm, src_v, dst_v,
                 rows_a, rows_b, rows_c, rows_d, stage, acc,
                 sem_a, sem_b, sem_c, sem_d):
    c = lax.axis_index("c")
    s = lax.axis_index("s")
    w = s * 2 + c

    @pl.when(s < 10)
    def _():
        pltpu.sync_copy(z_hbm, stage)
        pltpu.sync_copy(stage, acc.at[pl.ds(s * 1000, 1000)])

    @pl.when(s == 10)
    def _():
        pltpu.sync_copy(z_hbm, stage)
        pltpu.sync_copy(stage.at[pl.ds(0, K_JUNK)], acc.at[pl.ds(N, K_JUNK)])

    pltpu.sync_copy(src_hbm.at[pl.ds(w * CPW, CPW)], src_v)
    pltpu.sync_copy(dst_hbm.at[pl.ds(w * CPW, CPW)], dst_v)
    plsc.subcore_barrier()

    bufs = (rows_a, rows_b, rows_c, rows_d)
    sems = (sem_a, sem_b, sem_c, sem_d)

    def body(jj, carry):
        j0 = 4 * jj
        cps = [pltpu.async_copy(g_hbm.at[src_v.at[j0 + b]], bufs[b], sems[b])
               for b in range(4)]
        for b in range(4):
            cps[b].wait()
            pltpu.sync_copy(bufs[b], acc.at[dst_v.at[j0 + b]], add=True)
        return carry

    lax.fori_loop(0, CPW // 4, body, 0)
    plsc.subcore_barrier()

    @pl.when(s < 10)
    def _():
        pltpu.sync_copy(acc.at[pl.ds(s * 1000, 1000)], stage)
        pltpu.sync_copy(stage, out_hbm.at[c].at[pl.ds(s * 1000, 1000)])


@functools.cache
def _sc_kernels():
    mesh = plsc.VectorSubcoreMesh(core_axis_name="c", subcore_axis_name="s",
                                  num_cores=2, num_subcores=16)
    params = pltpu.CompilerParams(use_tc_tiling_on_sc=False)
    sc_deg = pl.kernel(
        _sc_deg_body,
        compiler_params=params,
        out_type=jax.ShapeDtypeStruct((2, 10, 1000), jnp.float32),
        mesh=mesh,
        scratch_types=[
            pltpu.VMEM((CPW, CH), jnp.int32),
            pltpu.VMEM((CH,), jnp.float32),
            pltpu.VMEM((1000,), jnp.float32),
            pltpu.VMEM_SHARED((NP,), jnp.float32),
        ],
    )
    sc_agg = pl.kernel(
        _sc_agg_body,
        compiler_params=params,
        out_type=jax.ShapeDtypeStruct((2, N, H), jnp.float32),
        mesh=mesh,
        scratch_types=[
            pltpu.VMEM((CPW, CH), jnp.int32),
            pltpu.VMEM((CPW, CH), jnp.int32),
            pltpu.VMEM((CH, H), jnp.float32),
            pltpu.VMEM((CH, H), jnp.float32),
            pltpu.VMEM((CH, H), jnp.float32),
            pltpu.VMEM((CH, H), jnp.float32),
            pltpu.VMEM((1000, H), jnp.float32),
            pltpu.VMEM_SHARED((NP, H), jnp.float32),
            pltpu.SemaphoreType.DMA,
            pltpu.SemaphoreType.DMA,
            pltpu.SemaphoreType.DMA,
            pltpu.SemaphoreType.DMA,
        ],
    )
    return sc_deg, sc_agg


def _tc1_body(x_ref, w1_ref, degt_ref, h1_ref, dinv_ref, g1_ref):
    h1 = jnp.dot(x_ref[...], w1_ref[...], preferred_element_type=jnp.float32)
    deg = degt_ref[:, 0:1] + degt_ref[:, 1:2] + 1.0
    dinv = lax.rsqrt(deg)
    h1_ref[...] = h1
    dinv_ref[...] = dinv
    g1_ref[...] = dinv * h1


def _tc2_body(aggA_ref, aggB_ref, h1_ref, dinv_ref, b1_ref, out1_ref, st_ref):
    i = pl.program_id(0)
    dv = dinv_ref[...]
    o = dv * (aggA_ref[...] + aggB_ref[...]) + dv * dv * h1_ref[...] + b1_ref[...]
    out1_ref[...] = o
    ps = jnp.sum(o, axis=0, keepdims=True)
    pq = jnp.sum(o * o, axis=0, keepdims=True)
    pblk = jnp.concatenate([ps, pq], axis=0)

    @pl.when(i == 0)
    def _():
        st_ref[...] = pblk

    @pl.when(i > 0)
    def _():
        st_ref[...] = st_ref[...] + pblk


def _tc3_body(out1_ref, st_ref, gamma_ref, beta_ref, w2_ref, dinv_ref,
              h2_ref, g2_ref):
    st = st_ref[...]
    mean = st[0:1, :] * (1.0 / N)
    var = st[1:2, :] * (1.0 / N) - mean * mean
    z = (out1_ref[...] - mean) * lax.rsqrt(var + 1e-5) * gamma_ref[...] + beta_ref[...]
    z = jnp.maximum(z, 0.0)
    h2 = jnp.dot(z, w2_ref[...], preferred_element_type=jnp.float32)
    h2_ref[...] = h2
    g2_ref[...] = dinv_ref[...] * h2


def _tc4_body(aggA_ref, aggB_ref, h2_ref, dinv_ref, b2_ref, out_ref):
    dv = dinv_ref[...]
    o = dv * (aggA_ref[...] + aggB_ref[...]) + dv * dv * h2_ref[...] + b2_ref[...]
    o0 = o[:, 0:1]
    o1 = o[:, 1:2]
    m = jnp.maximum(o0, o1)
    lse = m + jnp.log(jnp.exp(o0 - m) + jnp.exp(o1 - m))
    out_ref[...] = jnp.concatenate([o0, o1], axis=1) - lse


def _row_spec(width):
    return pl.BlockSpec((BLK, width), lambda i: (i, 0))


def _full_spec(shape):
    return pl.BlockSpec(shape, lambda i: (0,) * len(shape))


def kernel(x, edge_index, W1, b1, gamma, beta, W2, b2):
    src = edge_index[0].astype(jnp.int32)
    dst = edge_index[1].astype(jnp.int32)
    ar = jnp.arange(PAD, dtype=jnp.int32)
    src_p = jnp.concatenate([src, (ar * 131) % N])
    dst_p = jnp.concatenate([dst, N + (ar % K_JUNK)])
    src2d = src_p.reshape(ROWS2D, CH)
    dst2d = dst_p.reshape(ROWS2D, CH)

    zf = jnp.zeros((1000,), jnp.float32)
    z16 = jnp.zeros((1000, H), jnp.float32)

    sc_deg, sc_agg = _sc_kernels()

    degp = sc_deg(dst2d, zf)
    degt = degp.reshape(2, N).T

    h1, dinv, g1 = pl.pallas_call(
        _tc1_body,
        grid=(GRID,),
        in_specs=[_row_spec(D), _full_spec((D, H)), _row_spec(2)],
        out_specs=[_row_spec(H), _row_spec(1), _row_spec(H)],
        out_shape=[
            jax.ShapeDtypeStruct((N, H), jnp.float32),
            jax.ShapeDtypeStruct((N, 1), jnp.float32),
            jax.ShapeDtypeStruct((N, H), jnp.float32),
        ],
    )(x, W1, degt)

    agg1 = sc_agg(src2d, dst2d, g1, z16)

    out1, st = pl.pallas_call(
        _tc2_body,
        grid=(GRID,),
        in_specs=[_row_spec(H), _row_spec(H), _row_spec(H), _row_spec(1),
                  _full_spec((1, H))],
        out_specs=[_row_spec(H), _full_spec((2, H))],
        out_shape=[
            jax.ShapeDtypeStruct((N, H), jnp.float32),
            jax.ShapeDtypeStruct((2, H), jnp.float32),
        ],
    )(agg1[0], agg1[1], h1, dinv, b1.reshape(1, H))

    W2p = jnp.pad(W2, ((0, 0), (0, H - C)))
    h2, g2 = pl.pallas_call(
        _tc3_body,
        grid=(GRID,),
        in_specs=[_row_spec(H), _full_spec((2, H)), _full_spec((1, H)),
                  _full_spec((1, H)), _full_spec((H, H)), _row_spec(1)],
        out_specs=[_row_spec(H), _row_spec(H)],
        out_shape=[
            jax.ShapeDtypeStruct((N, H), jnp.float32),
            jax.ShapeDtypeStruct((N, H), jnp.float32),
        ],
    )(out1, st, gamma.reshape(1, H), beta.reshape(1, H), W2p, dinv)

    agg2 = sc_agg(src2d, dst2d, g2, z16)

    b2p = jnp.pad(b2, (0, H - C)).reshape(1, H)
    out = pl.pallas_call(
        _tc4_body,
        grid=(GRID,),
        in_specs=[_row_spec(H), _row_spec(H), _row_spec(H), _row_spec(1),
                  _full_spec((1, H))],
        out_specs=_row_spec(C),
        out_shape=jax.ShapeDtypeStruct((N, C), jnp.float32),
    )(agg2[0], agg2[1], h2, dinv, b2p)
    return out


# speedup vs baseline: 46.1415x; 1.0238x over previous
import functools

import jax
import jax.numpy as jnp
from jax import lax
from jax.experimental import pallas as pl
from jax.experimental.pallas import tpu as pltpu
from jax.experimental.pallas import tpu_sc as plsc

N = 10000
E = 320000
D = 128
H = 16
C = 2

NW = 32
CH = 128
CPW = 80
EP = NW * CPW * CH
PAD = EP - E
ROWS2D = EP // CH
K_JUNK = 16
NP = N + K_JUNK

BLK = 2000
GRID = N // BLK

def _sc_deg_body(dst_hbm, zf_hbm, out_hbm, dst_v, ones_v, stage, acc):
    c = lax.axis_index("c")
    s = lax.axis_index("s")
    w = s * 2 + c

    @pl.when(s < 10)
    def _():
        pltpu.sync_copy(zf_hbm, stage)
        pltpu.sync_copy(stage, acc.at[pl.ds(s * 1000, 1000)])

    @pl.when(s == 10)
    def _():
        pltpu.sync_copy(zf_hbm, stage)
        pltpu.sync_copy(stage.at[pl.ds(0, K_JUNK)], acc.at[pl.ds(N, K_JUNK)])

    pltpu.sync_copy(dst_hbm.at[pl.ds(w * CPW, CPW)], dst_v)
    for i in range(CH // 16):
        ones_v[pl.ds(i * 16, 16)] = jnp.ones((16,), jnp.float32)
    plsc.subcore_barrier()

    def body(j, carry):
        pltpu.sync_copy(ones_v, acc.at[dst_v.at[j]], add=True)
        return carry

    lax.fori_loop(0, CPW, body, 0)
    plsc.subcore_barrier()

    @pl.when(s < 10)
    def _():
        pltpu.sync_copy(acc.at[pl.ds(s * 1000, 1000)], stage)
        pltpu.sync_copy(stage, out_hbm.at[c].at[s])


def _sc_agg_body(src_hbm, dst_hbm, g_hbm, z_hbm, out_hbm, src_v, dst_v,
                 rows_a, rows_b, rows_c, rows_d, stage, acc,
                 sem_a, sem_b, sem_c, sem_d):
    c = lax.axis_index("c")
    s = lax.axis_index("s")
    w = s * 2 + c

    @pl.when(s < 10)
    def _():
        pltpu.sync_copy(z_hbm, stage)
        pltpu.sync_copy(stage, acc.at[pl.ds(s * 1000, 1000)])

    @pl.when(s == 10)
    def _():
        pltpu.sync_copy(z_hbm, stage)
        pltpu.sync_copy(stage.at[pl.ds(0, K_JUNK)], acc.at[pl.ds(N, K_JUNK)])

    pltpu.sync_copy(src_hbm.at[pl.ds(w * CPW, CPW)], src_v)
    pltpu.sync_copy(dst_hbm.at[pl.ds(w * CPW, CPW)], dst_v)
    plsc.subcore_barrier()

    bufs = (rows_a, rows_b, rows_c, rows_d)
    sems = (sem_a, sem_b, sem_c, sem_d)

    def body(jj, carry):
        j0 = 4 * jj
        cps = [pltpu.async_copy(g_hbm.at[src_v.at[j0 + b]], bufs[b], sems[b])
               for b in range(4)]
        for b in range(4):
            cps[b].wait()
            pltpu.sync_copy(bufs[b], acc.at[dst_v.at[j0 + b]], add=True)
        return carry

    lax.fori_loop(0, CPW // 4, body, 0)
    plsc.subcore_barrier()

    @pl.when(s < 10)
    def _():
        pltpu.sync_copy(acc.at[pl.ds(s * 1000, 1000)], stage)
        pltpu.sync_copy(stage, out_hbm.at[c].at[pl.ds(s * 1000, 1000)])


@functools.cache
def _sc_kernels():
    mesh = plsc.VectorSubcoreMesh(core_axis_name="c", subcore_axis_name="s",
                                  num_cores=2, num_subcores=16)
    params = pltpu.CompilerParams(use_tc_tiling_on_sc=False)
    sc_deg = pl.kernel(
        _sc_deg_body,
        compiler_params=params,
        out_type=jax.ShapeDtypeStruct((2, 10, 1000), jnp.float32),
        mesh=mesh,
        scratch_types=[
            pltpu.VMEM((CPW, CH), jnp.int32),
            pltpu.VMEM((CH,), jnp.float32),
            pltpu.VMEM((1000,), jnp.float32),
            pltpu.VMEM_SHARED((NP,), jnp.float32),
        ],
    )
    sc_agg = pl.kernel(
        _sc_agg_body,
        compiler_params=params,
        out_type=jax.ShapeDtypeStruct((2, N, H), jnp.float32),
        mesh=mesh,
        scratch_types=[
            pltpu.VMEM((CPW, CH), jnp.int32),
            pltpu.VMEM((CPW, CH), jnp.int32),
            pltpu.VMEM((CH, H), jnp.float32),
            pltpu.VMEM((CH, H), jnp.float32),
            pltpu.VMEM((CH, H), jnp.float32),
            pltpu.VMEM((CH, H), jnp.float32),
            pltpu.VMEM((1000, H), jnp.float32),
            pltpu.VMEM_SHARED((NP, H), jnp.float32),
            pltpu.SemaphoreType.DMA,
            pltpu.SemaphoreType.DMA,
            pltpu.SemaphoreType.DMA,
            pltpu.SemaphoreType.DMA,
        ],
    )
    return sc_deg, sc_agg


def _tc1_body(x_ref, w1_ref, degt_ref, h1_ref, dinv_ref, g1_ref):
    h1 = jnp.dot(x_ref[...], w1_ref[...], preferred_element_type=jnp.float32)
    deg = degt_ref[:, 0:1] + degt_ref[:, 1:2] + 1.0
    dinv = lax.rsqrt(deg)
    h1_ref[...] = h1
    dinv_ref[...] = dinv
    g1_ref[...] = dinv * h1


def _tc23_body(aggA_ref, aggB_ref, h1_ref, dinv_ref, b1_ref, gamma_ref,
               beta_ref, w2_ref, h2_ref, g2_ref):
    dv = dinv_ref[...]
    o = dv * (aggA_ref[...] + aggB_ref[...]) + dv * dv * h1_ref[...] + b1_ref[...]
    mean = jnp.mean(o, axis=0, keepdims=True)
    var = jnp.mean(o * o, axis=0, keepdims=True) - mean * mean
    z = (o - mean) * lax.rsqrt(var + 1e-5) * gamma_ref[...] + beta_ref[...]
    z = jnp.maximum(z, 0.0)
    h2 = jnp.dot(z, w2_ref[...], preferred_element_type=jnp.float32)
    h2_ref[...] = h2
    g2_ref[...] = dv * h2


def _tc4_body(aggA_ref, aggB_ref, h2_ref, dinv_ref, b2_ref, out_ref):
    dv = dinv_ref[...]
    o = dv * (aggA_ref[...] + aggB_ref[...]) + dv * dv * h2_ref[...] + b2_ref[...]
    o0 = o[:, 0:1]
    o1 = o[:, 1:2]
    m = jnp.maximum(o0, o1)
    lse = m + jnp.log(jnp.exp(o0 - m) + jnp.exp(o1 - m))
    out_ref[...] = jnp.concatenate([o0, o1], axis=1) - lse


def _row_spec(width):
    return pl.BlockSpec((BLK, width), lambda i: (i, 0))


def _full_spec(shape):
    return pl.BlockSpec(shape, lambda i: (0,) * len(shape))


def kernel(x, edge_index, W1, b1, gamma, beta, W2, b2):
    src = edge_index[0].astype(jnp.int32)
    dst = edge_index[1].astype(jnp.int32)
    ar = jnp.arange(PAD, dtype=jnp.int32)
    src_p = jnp.concatenate([src, (ar * 131) % N])
    dst_p = jnp.concatenate([dst, N + (ar % K_JUNK)])
    src2d = src_p.reshape(ROWS2D, CH)
    dst2d = dst_p.reshape(ROWS2D, CH)

    zf = jnp.zeros((1000,), jnp.float32)
    z16 = jnp.zeros((1000, H), jnp.float32)

    sc_deg, sc_agg = _sc_kernels()

    degp = sc_deg(dst2d, zf)
    degt = degp.reshape(2, N).T

    h1, dinv, g1 = pl.pallas_call(
        _tc1_body,
        grid=(GRID,),
        in_specs=[_row_spec(D), _full_spec((D, H)), _row_spec(2)],
        out_specs=[_row_spec(H), _row_spec(1), _row_spec(H)],
        out_shape=[
            jax.ShapeDtypeStruct((N, H), jnp.float32),
            jax.ShapeDtypeStruct((N, 1), jnp.float32),
            jax.ShapeDtypeStruct((N, H), jnp.float32),
        ],
    )(x, W1, degt)

    agg1 = sc_agg(src2d, dst2d, g1, z16)

    W2p = jnp.pad(W2, ((0, 0), (0, H - C)))
    h2, g2 = pl.pallas_call(
        _tc23_body,
        grid=(1,),
        in_specs=[_full_spec((N, H)), _full_spec((N, H)), _full_spec((N, H)),
                  _full_spec((N, 1)), _full_spec((1, H)), _full_spec((1, H)),
                  _full_spec((1, H)), _full_spec((H, H))],
        out_specs=[_full_spec((N, H)), _full_spec((N, H))],
        out_shape=[
            jax.ShapeDtypeStruct((N, H), jnp.float32),
            jax.ShapeDtypeStruct((N, H), jnp.float32),
        ],
    )(agg1[0], agg1[1], h1, dinv, b1.reshape(1, H), gamma.reshape(1, H),
      beta.reshape(1, H), W2p)

    agg2 = sc_agg(src2d, dst2d, g2, z16)

    b2p = jnp.pad(b2, (0, H - C)).reshape(1, H)
    out = pl.pallas_call(
        _tc4_body,
        grid=(GRID,),
        in_specs=[_row_spec(H), _row_spec(H), _row_spec(H), _row_spec(1),
                  _full_spec((1, H))],
        out_specs=_row_spec(C),
        out_shape=jax.ShapeDtypeStruct((N, C), jnp.float32),
    )(agg2[0], agg2[1], h2, dinv, b2p)
    return out


# speedup vs baseline: 52.8874x; 1.1462x over previous
import functools

import jax
import jax.numpy as jnp
from jax import lax
from jax.experimental import pallas as pl
from jax.experimental.pallas import tpu as pltpu
from jax.experimental.pallas import tpu_sc as plsc

N = 10000
E = 320000
D = 128
H = 16
C = 2

NW = 32
CH = 128
CPW = 80
EP = NW * CPW * CH
PAD = EP - E
ROWS2D = EP // CH
K_JUNK = 16
NP = N + K_JUNK

BLK = 2000
GRID = N // BLK
NBUF = 8

def _sc_deg_body(dst_hbm, zf_hbm, out_hbm, dst_v, ones_v, stage, acc, *sems):
    c = lax.axis_index("c")
    s = lax.axis_index("s")
    w = s * 2 + c

    @pl.when(s < 10)
    def _():
        pltpu.sync_copy(zf_hbm, stage)
        pltpu.sync_copy(stage, acc.at[pl.ds(s * 1000, 1000)])

    @pl.when(s == 10)
    def _():
        pltpu.sync_copy(zf_hbm, stage)
        pltpu.sync_copy(stage.at[pl.ds(0, K_JUNK)], acc.at[pl.ds(N, K_JUNK)])

    pltpu.sync_copy(dst_hbm.at[pl.ds(w * CPW, CPW)], dst_v)
    for i in range(CH // 16):
        ones_v[pl.ds(i * 16, 16)] = jnp.ones((16,), jnp.float32)
    plsc.subcore_barrier()

    def body(jj, carry):
        j0 = NBUF * jj
        cps = [pltpu.async_copy(ones_v, acc.at[dst_v.at[j0 + b]], sems[b],
                                add=True)
               for b in range(NBUF)]
        for cp in cps:
            cp.wait()
        return carry

    lax.fori_loop(0, CPW // NBUF, body, 0)
    plsc.subcore_barrier()

    @pl.when(s < 10)
    def _():
        pltpu.sync_copy(acc.at[pl.ds(s * 1000, 1000)], stage)
        pltpu.sync_copy(stage, out_hbm.at[c].at[s])


def _sc_agg_body(src_hbm, dst_hbm, g_hbm, z_hbm, out_hbm, src_v, dst_v,
                 stage, acc, *bufs_sems):
    bufs = bufs_sems[:NBUF]
    gsems = bufs_sems[NBUF:2 * NBUF]
    ssems = bufs_sems[2 * NBUF:]
    c = lax.axis_index("c")
    s = lax.axis_index("s")
    w = s * 2 + c

    @pl.when(s < 10)
    def _():
        pltpu.sync_copy(z_hbm, stage)
        pltpu.sync_copy(stage, acc.at[pl.ds(s * 1000, 1000)])

    @pl.when(s == 10)
    def _():
        pltpu.sync_copy(z_hbm, stage)
        pltpu.sync_copy(stage.at[pl.ds(0, K_JUNK)], acc.at[pl.ds(N, K_JUNK)])

    pltpu.sync_copy(src_hbm.at[pl.ds(w * CPW, CPW)], src_v)
    pltpu.sync_copy(dst_hbm.at[pl.ds(w * CPW, CPW)], dst_v)
    plsc.subcore_barrier()

    def body(jj, carry):
        j0 = NBUF * jj
        gcps = [pltpu.async_copy(g_hbm.at[src_v.at[j0 + b]], bufs[b], gsems[b])
                for b in range(NBUF)]
        scps = []
        for b in range(NBUF):
            gcps[b].wait()
            scps.append(pltpu.async_copy(bufs[b], acc.at[dst_v.at[j0 + b]],
                                         ssems[b], add=True))
        for cp in scps:
            cp.wait()
        return carry

    lax.fori_loop(0, CPW // NBUF, body, 0)
    plsc.subcore_barrier()

    @pl.when(s < 10)
    def _():
        pltpu.sync_copy(acc.at[pl.ds(s * 1000, 1000)], stage)
        pltpu.sync_copy(stage, out_hbm.at[c].at[pl.ds(s * 1000, 1000)])


@functools.cache
def _sc_kernels():
    mesh = plsc.VectorSubcoreMesh(core_axis_name="c", subcore_axis_name="s",
                                  num_cores=2, num_subcores=16)
    params = pltpu.CompilerParams(use_tc_tiling_on_sc=False)
    sc_deg = pl.kernel(
        _sc_deg_body,
        compiler_params=params,
        out_type=jax.ShapeDtypeStruct((2, 10, 1000), jnp.float32),
        mesh=mesh,
        scratch_types=[
            pltpu.VMEM((CPW, CH), jnp.int32),
            pltpu.VMEM((CH,), jnp.float32),
            pltpu.VMEM((1000,), jnp.float32),
            pltpu.VMEM_SHARED((NP,), jnp.float32),
        ] + [pltpu.SemaphoreType.DMA] * NBUF,
    )
    sc_agg = pl.kernel(
        _sc_agg_body,
        compiler_params=params,
        out_type=jax.ShapeDtypeStruct((2, N, H), jnp.float32),
        mesh=mesh,
        scratch_types=[
            pltpu.VMEM((CPW, CH), jnp.int32),
            pltpu.VMEM((CPW, CH), jnp.int32),
            pltpu.VMEM((1000, H), jnp.float32),
            pltpu.VMEM_SHARED((NP, H), jnp.float32),
        ] + [pltpu.VMEM((CH, H), jnp.float32)] * NBUF
          + [pltpu.SemaphoreType.DMA] * (2 * NBUF),
    )
    return sc_deg, sc_agg


def _tc1_body(x_ref, w1_ref, degt_ref, h1_ref, dinv_ref, g1_ref):
    h1 = jnp.dot(x_ref[...], w1_ref[...], preferred_element_type=jnp.float32)
    deg = degt_ref[:, 0:1] + degt_ref[:, 1:2] + 1.0
    dinv = lax.rsqrt(deg)
    h1_ref[...] = h1
    dinv_ref[...] = dinv
    g1_ref[...] = dinv * h1


def _tc23_body(aggA_ref, aggB_ref, h1_ref, dinv_ref, b1_ref, gamma_ref,
               beta_ref, w2_ref, h2_ref, g2_ref):
    dv = dinv_ref[...]
    o = dv * (aggA_ref[...] + aggB_ref[...]) + dv * dv * h1_ref[...] + b1_ref[...]
    mean = jnp.mean(o, axis=0, keepdims=True)
    var = jnp.mean(o * o, axis=0, keepdims=True) - mean * mean
    z = (o - mean) * lax.rsqrt(var + 1e-5) * gamma_ref[...] + beta_ref[...]
    z = jnp.maximum(z, 0.0)
    h2 = jnp.dot(z, w2_ref[...], preferred_element_type=jnp.float32)
    h2_ref[...] = h2
    g2_ref[...] = dv * h2


def _tc4_body(aggA_ref, aggB_ref, h2_ref, dinv_ref, b2_ref, out_ref):
    dv = dinv_ref[...]
    o = dv * (aggA_ref[...] + aggB_ref[...]) + dv * dv * h2_ref[...] + b2_ref[...]
    o0 = o[:, 0:1]
    o1 = o[:, 1:2]
    m = jnp.maximum(o0, o1)
    lse = m + jnp.log(jnp.exp(o0 - m) + jnp.exp(o1 - m))
    out_ref[...] = jnp.concatenate([o0, o1], axis=1) - lse


def _row_spec(width):
    return pl.BlockSpec((BLK, width), lambda i: (i, 0))


def _full_spec(shape):
    return pl.BlockSpec(shape, lambda i: (0,) * len(shape))


def kernel(x, edge_index, W1, b1, gamma, beta, W2, b2):
    src = edge_index[0].astype(jnp.int32)
    dst = edge_index[1].astype(jnp.int32)
    ar = jnp.arange(PAD, dtype=jnp.int32)
    src_p = jnp.concatenate([src, (ar * 131) % N])
    dst_p = jnp.concatenate([dst, N + (ar % K_JUNK)])
    src2d = src_p.reshape(ROWS2D, CH)
    dst2d = dst_p.reshape(ROWS2D, CH)

    zf = jnp.zeros((1000,), jnp.float32)
    z16 = jnp.zeros((1000, H), jnp.float32)

    sc_deg, sc_agg = _sc_kernels()

    degp = sc_deg(dst2d, zf)
    degt = degp.reshape(2, N).T

    h1, dinv, g1 = pl.pallas_call(
        _tc1_body,
        grid=(GRID,),
        in_specs=[_row_spec(D), _full_spec((D, H)), _row_spec(2)],
        out_specs=[_row_spec(H), _row_spec(1), _row_spec(H)],
        out_shape=[
            jax.ShapeDtypeStruct((N, H), jnp.float32),
            jax.ShapeDtypeStruct((N, 1), jnp.float32),
            jax.ShapeDtypeStruct((N, H), jnp.float32),
        ],
    )(x, W1, degt)

    agg1 = sc_agg(src2d, dst2d, g1, z16)

    W2p = jnp.pad(W2, ((0, 0), (0, H - C)))
    h2, g2 = pl.pallas_call(
        _tc23_body,
        grid=(1,),
        in_specs=[_full_spec((N, H)), _full_spec((N, H)), _full_spec((N, H)),
                  _full_spec((N, 1)), _full_spec((1, H)), _full_spec((1, H)),
                  _full_spec((1, H)), _full_spec((H, H))],
        out_specs=[_full_spec((N, H)), _full_spec((N, H))],
        out_shape=[
            jax.ShapeDtypeStruct((N, H), jnp.float32),
            jax.ShapeDtypeStruct((N, H), jnp.float32),
        ],
    )(agg1[0], agg1[1], h1, dinv, b1.reshape(1, H), gamma.reshape(1, H),
      beta.reshape(1, H), W2p)

    agg2 = sc_agg(src2d, dst2d, g2, z16)

    b2p = jnp.pad(b2, (0, H - C)).reshape(1, H)
    out = pl.pallas_call(
        _tc4_body,
        grid=(GRID,),
        in_specs=[_row_spec(H), _row_spec(H), _row_spec(H), _row_spec(1),
                  _full_spec((1, H))],
        out_specs=_row_spec(C),
        out_shape=jax.ShapeDtypeStruct((N, C), jnp.float32),
    )(agg2[0], agg2[1], h2, dinv, b2p)
    return out


# speedup vs baseline: 52.9931x; 1.0020x over previous
import functools

import jax
import jax.numpy as jnp
from jax import lax
from jax.experimental import pallas as pl
from jax.experimental.pallas import tpu as pltpu
from jax.experimental.pallas import tpu_sc as plsc

N = 10000
E = 320000
D = 128
H = 16
C = 2

NW = 32
CH = 128
CPW = 80
EP = NW * CPW * CH
PAD = EP - E
ROWS2D = EP // CH
K_JUNK = 16
NP = N + K_JUNK

BLK = 2000
GRID = N // BLK
NBUF = 8

def _sc_deg_body(dst_hbm, zf_hbm, out_hbm, dst_v, ones_v, stage, acc, *sems):
    c = lax.axis_index("c")
    s = lax.axis_index("s")
    w = s * 2 + c

    @pl.when(s < 10)
    def _():
        pltpu.sync_copy(zf_hbm, stage)
        pltpu.sync_copy(stage, acc.at[pl.ds(s * 1000, 1000)])

    @pl.when(s == 10)
    def _():
        pltpu.sync_copy(zf_hbm, stage)
        pltpu.sync_copy(stage.at[pl.ds(0, K_JUNK)], acc.at[pl.ds(N, K_JUNK)])

    pltpu.sync_copy(dst_hbm.at[pl.ds(w * CPW, CPW)], dst_v)
    for i in range(CH // 16):
        ones_v[pl.ds(i * 16, 16)] = jnp.ones((16,), jnp.float32)
    plsc.subcore_barrier()

    def body(jj, carry):
        j0 = NBUF * jj
        cps = [pltpu.async_copy(ones_v, acc.at[dst_v.at[j0 + b]], sems[b],
                                add=True)
               for b in range(NBUF)]
        for cp in cps:
            cp.wait()
        return carry

    lax.fori_loop(0, CPW // NBUF, body, 0)
    plsc.subcore_barrier()

    @pl.when(s < 10)
    def _():
        pltpu.sync_copy(acc.at[pl.ds(s * 1000, 1000)], stage)
        pltpu.sync_copy(stage, out_hbm.at[c].at[s])


def _rsqrt16(d):
    i = plsc.bitcast(d, jnp.int32)
    i = jnp.int32(0x5F3759DF) - (i >> 1)
    y = plsc.bitcast(i, jnp.float32)
    for _ in range(4):
        y = y * (1.5 - 0.5 * d * y * y)
    return y


def _sc_agg_body(src_hbm, dst_hbm, degp_hbm, feat_hbm, z_hbm,
                 out_hbm, gtab_hbm, src_v, dst_v, stage, acc,
                 degA_v, degB_v, dinv_v, feat_v, g_v, *bufs_sems):
    bufs = bufs_sems[:NBUF]
    gsems = bufs_sems[NBUF:2 * NBUF]
    ssems = bufs_sems[2 * NBUF:]
    c = lax.axis_index("c")
    s = lax.axis_index("s")
    w = s * 2 + c
    iota = lax.iota(jnp.int32, 16)

    @pl.when(s < 10)
    def _():
        pltpu.sync_copy(z_hbm, stage)
        pltpu.sync_copy(stage, acc.at[pl.ds(s * 1000, 1000)])

    @pl.when(s == 10)
    def _():
        pltpu.sync_copy(z_hbm, stage)
        pltpu.sync_copy(stage.at[pl.ds(0, K_JUNK)], acc.at[pl.ds(N, K_JUNK)])

    pltpu.sync_copy(src_hbm.at[pl.ds(w * CPW, CPW)], src_v)
    pltpu.sync_copy(dst_hbm.at[pl.ds(w * CPW, CPW)], dst_v)

    @pl.when(s < 10)
    def _():
        pltpu.sync_copy(degp_hbm.at[0].at[s], degA_v.at[pl.ds(0, 1000)])
        pltpu.sync_copy(degp_hbm.at[1].at[s], degB_v.at[pl.ds(0, 1000)])
        pltpu.sync_copy(feat_hbm.at[pl.ds(s * 1000, 1000)], feat_v)

        def rbody(k, carry):
            idx = k * 16 + iota
            d = (plsc.load_gather(degA_v, [idx])
                 + plsc.load_gather(degB_v, [idx]) + 1.0)
            plsc.store_scatter(dinv_v, [idx], _rsqrt16(d))
            return carry

        lax.fori_loop(0, 63, rbody, 0)

        def gbody(i, carry):
            fi = jnp.full((16,), i, jnp.int32)
            dv = plsc.load_gather(dinv_v, [fi])
            row = plsc.load_gather(feat_v, [fi, iota])
            plsc.store_scatter(g_v, [fi, iota], row * dv)
            return carry

        lax.fori_loop(0, 1000, gbody, 0)
        pltpu.sync_copy(g_v, gtab_hbm.at[pl.ds(s * 1000, 1000)])

    plsc.subcore_barrier()

    def body(jj, carry):
        j0 = NBUF * jj
        gcps = [pltpu.async_copy(gtab_hbm.at[src_v.at[j0 + b]], bufs[b],
                                 gsems[b])
                for b in range(NBUF)]
        scps = []
        for b in range(NBUF):
            gcps[b].wait()
            scps.append(pltpu.async_copy(bufs[b], acc.at[dst_v.at[j0 + b]],
                                         ssems[b], add=True))
        for cp in scps:
            cp.wait()
        return carry

    lax.fori_loop(0, CPW // NBUF, body, 0)
    plsc.subcore_barrier()

    @pl.when(s < 10)
    def _():
        pltpu.sync_copy(acc.at[pl.ds(s * 1000, 1000)], stage)
        cf = jnp.full((16,), (c == 0).astype(jnp.float32))

        def ebody(i, carry):
            fi = jnp.full((16,), i, jnp.int32)
            dv = plsc.load_gather(dinv_v, [fi])
            srow = plsc.load_gather(stage, [fi, iota])
            grow = plsc.load_gather(g_v, [fi, iota])
            plsc.store_scatter(stage, [fi, iota], (srow + grow * cf) * dv)
            return carry

        lax.fori_loop(0, 1000, ebody, 0)
        pltpu.sync_copy(stage, out_hbm.at[c].at[pl.ds(s * 1000, 1000)])


@functools.cache
def _sc_kernels():
    mesh = plsc.VectorSubcoreMesh(core_axis_name="c", subcore_axis_name="s",
                                  num_cores=2, num_subcores=16)
    params = pltpu.CompilerParams(use_tc_tiling_on_sc=False,
                                  needs_layout_passes=False)
    sc_deg = pl.kernel(
        _sc_deg_body,
        compiler_params=params,
        out_type=jax.ShapeDtypeStruct((2, 10, 1000), jnp.float32),
        mesh=mesh,
        scratch_types=[
            pltpu.VMEM((CPW, CH), jnp.int32),
            pltpu.VMEM((CH,), jnp.float32),
            pltpu.VMEM((1000,), jnp.float32),
            pltpu.VMEM_SHARED((NP,), jnp.float32),
        ] + [pltpu.SemaphoreType.DMA] * NBUF,
    )
    sc_agg = pl.kernel(
        _sc_agg_body,
        compiler_params=params,
        out_type=(jax.ShapeDtypeStruct((2, N, H), jnp.float32),
                  jax.ShapeDtypeStruct((N, H), jnp.float32)),
        mesh=mesh,
        scratch_types=[
            pltpu.VMEM((CPW, CH), jnp.int32),
            pltpu.VMEM((CPW, CH), jnp.int32),
            pltpu.VMEM((1000, H), jnp.float32),
            pltpu.VMEM_SHARED((NP, H), jnp.float32),
            pltpu.VMEM((1008,), jnp.float32),
            pltpu.VMEM((1008,), jnp.float32),
            pltpu.VMEM((1008,), jnp.float32),
            pltpu.VMEM((1000, H), jnp.float32),
            pltpu.VMEM((1000, H), jnp.float32),
        ] + [pltpu.VMEM((CH, H), jnp.float32)] * NBUF
          + [pltpu.SemaphoreType.DMA] * (2 * NBUF),
    )
    return sc_deg, sc_agg


def _tc1_body(x_ref, w1_ref, h1_ref):
    h1_ref[...] = jnp.dot(x_ref[...], w1_ref[...],
                          preferred_element_type=jnp.float32)


def _tc23_body(u_ref, b1_ref, gamma_ref, beta_ref, z_ref):
    o = u_ref[0] + u_ref[1] + b1_ref[...]
    mean = jnp.mean(o, axis=0, keepdims=True)
    var = jnp.mean(o * o, axis=0, keepdims=True) - mean * mean
    z = (o - mean) * lax.rsqrt(var + 1e-5) * gamma_ref[...] + beta_ref[...]
    z_ref[...] = jnp.maximum(z, 0.0)


def _tc4_body(u_ref, w2_ref, b2_ref, out_ref):
    o = jnp.dot(u_ref[0] + u_ref[1], w2_ref[...],
                preferred_element_type=jnp.float32) + b2_ref[...]
    o0 = o[:, 0:1]
    o1 = o[:, 1:2]
    m = jnp.maximum(o0, o1)
    lse = m + jnp.log(jnp.exp(o0 - m) + jnp.exp(o1 - m))
    out_ref[...] = jnp.concatenate([o0, o1], axis=1) - lse


def _row_spec(width):
    return pl.BlockSpec((BLK, width), lambda i: (i, 0))


def _full_spec(shape):
    return pl.BlockSpec(shape, lambda i: (0,) * len(shape))


def kernel(x, edge_index, W1, b1, gamma, beta, W2, b2):
    src = edge_index[0].astype(jnp.int32)
    dst = edge_index[1].astype(jnp.int32)
    ar = jnp.arange(PAD, dtype=jnp.int32)
    src_p = jnp.concatenate([src, (ar * 131) % N])
    dst_p = jnp.concatenate([dst, N + (ar % K_JUNK)])
    src2d = src_p.reshape(ROWS2D, CH)
    dst2d = dst_p.reshape(ROWS2D, CH)

    zf = jnp.zeros((1000,), jnp.float32)
    z16 = jnp.zeros((1000, H), jnp.float32)

    sc_deg, sc_agg = _sc_kernels()

    h1 = pl.pallas_call(
        _tc1_body,
        grid=(GRID,),
        in_specs=[_row_spec(D), _full_spec((D, H))],
        out_specs=_row_spec(H),
        out_shape=jax.ShapeDtypeStruct((N, H), jnp.float32),
    )(x, W1)

    degp = sc_deg(dst2d, zf)

    u1, _g1 = sc_agg(src2d, dst2d, degp, h1, z16)

    z = pl.pallas_call(
        _tc23_body,
        grid=(1,),
        in_specs=[_full_spec((2, N, H)), _full_spec((1, H)),
                  _full_spec((1, H)), _full_spec((1, H))],
        out_specs=_full_spec((N, H)),
        out_shape=jax.ShapeDtypeStruct((N, H), jnp.float32),
    )(u1, b1.reshape(1, H), gamma.reshape(1, H), beta.reshape(1, H))

    u2, _g2 = sc_agg(src2d, dst2d, degp, z, z16)

    W2p = jnp.pad(W2, ((0, 0), (0, H - C)))
    b2p = jnp.pad(b2, (0, H - C)).reshape(1, H)
    out = pl.pallas_call(
        _tc4_body,
        grid=(1,),
        in_specs=[_full_spec((2, N, H)), _full_spec((H, H)),
                  _full_spec((1, H))],
        out_specs=_full_spec((N, C)),
        out_shape=jax.ShapeDtypeStruct((N, C), jnp.float32),
    )(u2, W2p, b2p)
    return out
